# Initial kernel scaffold; baseline (speedup 1.0000x reference)
import jax
import jax.numpy as jnp
from jax.experimental import pallas as pl


def kernel(x, edge_index, W1l, b1, W1r, W2l, b2, W2r):
    raise NotImplementedError("write your pallas kernel here")


# speedup vs baseline: 8.5658x; 8.5658x over previous
import functools

import jax
import jax.numpy as jnp
from jax import lax
from jax.experimental import pallas as pl
from jax.experimental.pallas import tpu as pltpu
from jax.experimental.pallas import tpu_sc as plsc

N_NODES = 10000
N_EDGES = 320000
D_IN = 128
D_HID = 32
D_OUT = 128

NP_ = 10112
ZROW = N_NODES
TRASH = N_NODES + 8

NC, NS = 2, 16
NW = NC * NS
CH = 128
NCHUNK = 80
EPT = CH * NCHUNK
E_PAD = EPT * NW

R = NP_ // NS

_mesh = plsc.VectorSubcoreMesh(
    core_axis_name="c", subcore_axis_name="s", num_cores=NC, num_subcores=NS)


def _make_sc_pass(with_counts: bool):
    out_type = [jax.ShapeDtypeStruct((NC, NP_, 32), jnp.float32)]
    if with_counts:
        out_type.append(jax.ShapeDtypeStruct((NC, NP_, 16), jnp.float32))

    scratch = [
        pltpu.VMEM_SHARED((NP_, 32), jnp.float32),
        pltpu.VMEM((NCHUNK, CH), jnp.int32),
        pltpu.VMEM((NCHUNK, CH), jnp.int32),
        pltpu.VMEM((CH, 32), jnp.float32),
        pltpu.SemaphoreType.DMA,
        pltpu.SemaphoreType.DMA,
    ]
    if with_counts:
        scratch += [
            pltpu.VMEM_SHARED((NP_, 16), jnp.float32),
            pltpu.VMEM((CH, 16), jnp.float32),
            pltpu.SemaphoreType.DMA,
        ]

    def body(*refs):
        if with_counts:
            (table_h, src_h, dst_h, z32_h, z16_h, ones_h,
             parts_h, cntp_h,
             acc_s, src_v, dst_v, rows_v, gsem, ssem,
             cnt_s, ones_v, csem) = refs
        else:
            (table_h, src_h, dst_h, z32_h,
             parts_h,
             acc_s, src_v, dst_v, rows_v, gsem, ssem) = refs
        c = lax.axis_index("c")
        s = lax.axis_index("s")
        wid = c * NS + s

        pltpu.sync_copy(z32_h.at[pl.ds(s * R, R)], acc_s.at[pl.ds(s * R, R)])
        if with_counts:
            pltpu.sync_copy(z16_h.at[pl.ds(s * R, R)], cnt_s.at[pl.ds(s * R, R)])
            pltpu.sync_copy(ones_h, ones_v)
        pltpu.sync_copy(src_h.at[wid], src_v)
        pltpu.sync_copy(dst_h.at[wid], dst_v)
        plsc.subcore_barrier()

        def chunk(j, carry):
            pltpu.async_copy(table_h.at[src_v.at[j]], rows_v, gsem).wait()
            pltpu.async_copy(rows_v, acc_s.at[dst_v.at[j]], ssem, add=True).wait()
            if with_counts:
                pltpu.async_copy(ones_v, cnt_s.at[dst_v.at[j]], csem, add=True).wait()
            return carry

        lax.fori_loop(0, NCHUNK, chunk, 0)
        plsc.subcore_barrier()

        pltpu.sync_copy(acc_s.at[pl.ds(s * R, R)], parts_h.at[c, pl.ds(s * R, R)])
        if with_counts:
            pltpu.sync_copy(cnt_s.at[pl.ds(s * R, R)], cntp_h.at[c, pl.ds(s * R, R)])

    return pl.kernel(body, out_type=out_type, mesh=_mesh, scratch_types=scratch,
                     compiler_params=pltpu.CompilerParams(use_tc_tiling_on_sc=False))


_sc_pass1 = _make_sc_pass(with_counts=True)
_sc_pass2 = _make_sc_pass(with_counts=False)


_BR = 1000
_GRID = N_NODES // _BR


def _tc1_body(x_ref, w_ref, y_ref):
    y_ref[...] = jnp.dot(x_ref[...], w_ref[...],
                         preferred_element_type=jnp.float32)


def _tc1(x, w1catT):
    return pl.pallas_call(
        _tc1_body,
        grid=(_GRID,),
        in_specs=[
            pl.BlockSpec((_BR, D_IN), lambda i: (i, 0)),
            pl.BlockSpec((D_IN, 2 * D_HID), lambda i: (0, 0)),
        ],
        out_specs=pl.BlockSpec((_BR, 2 * D_HID), lambda i: (i, 0)),
        out_shape=jax.ShapeDtypeStruct((N_NODES, 2 * D_HID), jnp.float32),
    )(x, w1catT)


def _tc2_body(p_ref, cp_ref, r_ref, b_ref, h_ref):
    psum = p_ref[0] + p_ref[1]
    cnt = cp_ref[0, :, 0:1] + cp_ref[1, :, 0:1]
    inv = 1.0 / jnp.maximum(cnt, 1.0)
    h_ref[...] = jnp.maximum(psum * inv + b_ref[...] + r_ref[...], 0.0)


def _tc2(parts1, cntp, r1, b1):
    return pl.pallas_call(
        _tc2_body,
        grid=(_GRID,),
        in_specs=[
            pl.BlockSpec((NC, _BR, D_HID), lambda i: (0, i, 0)),
            pl.BlockSpec((NC, _BR, 16), lambda i: (0, i, 0)),
            pl.BlockSpec((_BR, D_HID), lambda i: (i, 0)),
            pl.BlockSpec((1, D_HID), lambda i: (0, 0)),
        ],
        out_specs=pl.BlockSpec((_BR, D_HID), lambda i: (i, 0)),
        out_shape=jax.ShapeDtypeStruct((N_NODES, D_HID), jnp.float32),
    )(parts1, cntp, r1, b1)


def _tc3_body(p_ref, cp_ref, h_ref, wl_ref, wr_ref, b_ref, o_ref):
    cnt = cp_ref[0, :, 0:1] + cp_ref[1, :, 0:1]
    inv = 1.0 / jnp.maximum(cnt, 1.0)
    agg = (p_ref[0] + p_ref[1]) * inv
    o_ref[...] = (
        jnp.dot(agg, wl_ref[...], preferred_element_type=jnp.float32)
        + jnp.dot(h_ref[...], wr_ref[...], preferred_element_type=jnp.float32)
        + b_ref[...])


def _tc3(parts2, cntp, h, w2lT, w2rT, b2):
    return pl.pallas_call(
        _tc3_body,
        grid=(_GRID,),
        in_specs=[
            pl.BlockSpec((NC, _BR, D_HID), lambda i: (0, i, 0)),
            pl.BlockSpec((NC, _BR, 16), lambda i: (0, i, 0)),
            pl.BlockSpec((_BR, D_HID), lambda i: (i, 0)),
            pl.BlockSpec((D_HID, D_OUT), lambda i: (0, 0)),
            pl.BlockSpec((D_HID, D_OUT), lambda i: (0, 0)),
            pl.BlockSpec((1, D_OUT), lambda i: (0, 0)),
        ],
        out_specs=pl.BlockSpec((_BR, D_OUT), lambda i: (i, 0)),
        out_shape=jax.ShapeDtypeStruct((N_NODES, D_OUT), jnp.float32),
    )(parts2, cntp, h, w2lT, w2rT, b2)


def kernel(x, edge_index, W1l, b1, W1r, W2l, b2, W2r):
    src = jnp.pad(edge_index[0].astype(jnp.int32), (0, E_PAD - N_EDGES),
                  constant_values=ZROW).reshape(NW, NCHUNK, CH)
    dst = jnp.pad(edge_index[1].astype(jnp.int32), (0, E_PAD - N_EDGES),
                  constant_values=TRASH).reshape(NW, NCHUNK, CH)

    z32 = jnp.zeros((NP_, 32), jnp.float32)
    z16 = jnp.zeros((NP_, 16), jnp.float32)
    ones = jnp.ones((CH, 16), jnp.float32)

    w1catT = jnp.concatenate([W1l, W1r], axis=0).T
    y = _tc1(x, w1catT)
    table1 = jnp.pad(y[:, :D_HID], ((0, NP_ - N_NODES), (0, 0)))
    r1 = y[:, D_HID:]

    parts1, cntp = _sc_pass1(table1, src, dst, z32, z16, ones)

    h = _tc2(parts1, cntp, r1, b1.reshape(1, D_HID))
    table2 = jnp.pad(h, ((0, NP_ - N_NODES), (0, 0)))

    (parts2,) = _sc_pass2(table2, src, dst, z32)

    return _tc3(parts2, cntp, h, W2l.T, W2r.T, b2.reshape(1, D_OUT))


# speedup vs baseline: 9.4526x; 1.1035x over previous
import functools

import jax
import jax.numpy as jnp
from jax import lax
from jax.experimental import pallas as pl
from jax.experimental.pallas import tpu as pltpu
from jax.experimental.pallas import tpu_sc as plsc

N_NODES = 10000
N_EDGES = 320000
D_IN = 128
D_HID = 32
D_OUT = 128

NP_ = 10112
ZROW = N_NODES
TRASH = N_NODES + 8

NC, NS = 2, 16
NW = NC * NS
CH = 128
NCHUNK = 80
EPT = CH * NCHUNK
E_PAD = EPT * NW

R = NP_ // NS

_mesh = plsc.VectorSubcoreMesh(
    core_axis_name="c", subcore_axis_name="s", num_cores=NC, num_subcores=NS)


def _make_sc_pass(with_counts: bool):
    out_type = [jax.ShapeDtypeStruct((NC, NP_, 32), jnp.float32)]
    if with_counts:
        out_type.append(jax.ShapeDtypeStruct((NC, NP_, 16), jnp.float32))

    scratch = [
        pltpu.VMEM_SHARED((NP_, 32), jnp.float32),
        pltpu.VMEM((NCHUNK, CH), jnp.int32),
        pltpu.VMEM((NCHUNK, CH), jnp.int32),
        pltpu.VMEM((CH, 32), jnp.float32),
        pltpu.VMEM((CH, 32), jnp.float32),
        pltpu.SemaphoreType.DMA,
        pltpu.SemaphoreType.DMA,
        pltpu.SemaphoreType.DMA,
        pltpu.SemaphoreType.DMA,
    ]
    if with_counts:
        scratch += [
            pltpu.VMEM_SHARED((NP_, 16), jnp.float32),
            pltpu.VMEM((CH, 16), jnp.float32),
            pltpu.SemaphoreType.DMA,
            pltpu.SemaphoreType.DMA,
        ]

    def body(*refs):
        if with_counts:
            (table_h, src_h, dst_h, z32_h, z16_h, ones_h,
             parts_h, cntp_h,
             acc_s, src_v, dst_v, rows0, rows1, gs0, gs1, ss0, ss1,
             cnt_s, ones_v, cs0, cs1) = refs
        else:
            (table_h, src_h, dst_h, z32_h,
             parts_h,
             acc_s, src_v, dst_v, rows0, rows1, gs0, gs1, ss0, ss1) = refs
        c = lax.axis_index("c")
        s = lax.axis_index("s")
        wid = c * NS + s

        pltpu.sync_copy(z32_h.at[pl.ds(s * R, R)], acc_s.at[pl.ds(s * R, R)])
        if with_counts:
            pltpu.sync_copy(z16_h.at[pl.ds(s * R, R)], cnt_s.at[pl.ds(s * R, R)])
            pltpu.sync_copy(ones_h, ones_v)
        pltpu.sync_copy(src_h.at[wid], src_v)
        pltpu.sync_copy(dst_h.at[wid], dst_v)
        plsc.subcore_barrier()

        def gather(j, buf, sem):
            return pltpu.async_copy(table_h.at[src_v.at[j]], buf, sem)

        def scat(j, buf, sem):
            return pltpu.async_copy(buf, acc_s.at[dst_v.at[j]], sem, add=True)

        def cnts(j, sem):
            return pltpu.async_copy(ones_v, cnt_s.at[dst_v.at[j]], sem, add=True)

        gather(0, rows0, gs0)

        def pair(i, carry):
            j0 = 2 * i
            j1 = j0 + 1
            pltpu.make_async_copy(table_h.at[src_v.at[j0]], rows0, gs0).wait()

            @pl.when(i > 0)
            def _():
                pltpu.make_async_copy(
                    rows1, acc_s.at[dst_v.at[j0]], ss1).wait()
                if with_counts:
                    pltpu.make_async_copy(
                        ones_v, cnt_s.at[dst_v.at[j0]], cs1).wait()

            g1 = gather(j1, rows1, gs1)
            s0 = scat(j0, rows0, ss0)
            c0 = cnts(j0, cs0) if with_counts else None
            g1.wait()
            s0.wait()
            if with_counts:
                c0.wait()

            @pl.when(i < NCHUNK // 2 - 1)
            def _():
                gather(j0 + 2, rows0, gs0)

            scat(j1, rows1, ss1)
            if with_counts:
                cnts(j1, cs1)
            return carry

        lax.fori_loop(0, NCHUNK // 2, pair, 0)
        pltpu.make_async_copy(rows1, acc_s.at[dst_v.at[NCHUNK - 1]], ss1).wait()
        if with_counts:
            pltpu.make_async_copy(
                ones_v, cnt_s.at[dst_v.at[NCHUNK - 1]], cs1).wait()
        plsc.subcore_barrier()

        pltpu.sync_copy(acc_s.at[pl.ds(s * R, R)], parts_h.at[c, pl.ds(s * R, R)])
        if with_counts:
            pltpu.sync_copy(cnt_s.at[pl.ds(s * R, R)], cntp_h.at[c, pl.ds(s * R, R)])

    return pl.kernel(body, out_type=out_type, mesh=_mesh, scratch_types=scratch,
                     compiler_params=pltpu.CompilerParams(use_tc_tiling_on_sc=False))


_sc_pass1 = _make_sc_pass(with_counts=True)
_sc_pass2 = _make_sc_pass(with_counts=False)


_BR = 1000
_GRID = N_NODES // _BR


def _tc1_body(x_ref, w_ref, y_ref):
    y_ref[...] = jnp.dot(x_ref[...], w_ref[...],
                         preferred_element_type=jnp.float32)


def _tc1(x, w1catT):
    return pl.pallas_call(
        _tc1_body,
        grid=(_GRID,),
        in_specs=[
            pl.BlockSpec((_BR, D_IN), lambda i: (i, 0)),
            pl.BlockSpec((D_IN, 2 * D_HID), lambda i: (0, 0)),
        ],
        out_specs=pl.BlockSpec((_BR, 2 * D_HID), lambda i: (i, 0)),
        out_shape=jax.ShapeDtypeStruct((N_NODES, 2 * D_HID), jnp.float32),
    )(x, w1catT)


def _tc2_body(p_ref, cp_ref, r_ref, b_ref, h_ref):
    psum = p_ref[0] + p_ref[1]
    cnt = cp_ref[0, :, 0:1] + cp_ref[1, :, 0:1]
    inv = 1.0 / jnp.maximum(cnt, 1.0)
    h_ref[...] = jnp.maximum(psum * inv + b_ref[...] + r_ref[...], 0.0)


def _tc2(parts1, cntp, r1, b1):
    return pl.pallas_call(
        _tc2_body,
        grid=(_GRID,),
        in_specs=[
            pl.BlockSpec((NC, _BR, D_HID), lambda i: (0, i, 0)),
            pl.BlockSpec((NC, _BR, 16), lambda i: (0, i, 0)),
            pl.BlockSpec((_BR, D_HID), lambda i: (i, 0)),
            pl.BlockSpec((1, D_HID), lambda i: (0, 0)),
        ],
        out_specs=pl.BlockSpec((_BR, D_HID), lambda i: (i, 0)),
        out_shape=jax.ShapeDtypeStruct((N_NODES, D_HID), jnp.float32),
    )(parts1, cntp, r1, b1)


def _tc3_body(p_ref, cp_ref, h_ref, wl_ref, wr_ref, b_ref, o_ref):
    cnt = cp_ref[0, :, 0:1] + cp_ref[1, :, 0:1]
    inv = 1.0 / jnp.maximum(cnt, 1.0)
    agg = (p_ref[0] + p_ref[1]) * inv
    o_ref[...] = (
        jnp.dot(agg, wl_ref[...], preferred_element_type=jnp.float32)
        + jnp.dot(h_ref[...], wr_ref[...], preferred_element_type=jnp.float32)
        + b_ref[...])


def _tc3(parts2, cntp, h, w2lT, w2rT, b2):
    return pl.pallas_call(
        _tc3_body,
        grid=(_GRID,),
        in_specs=[
            pl.BlockSpec((NC, _BR, D_HID), lambda i: (0, i, 0)),
            pl.BlockSpec((NC, _BR, 16), lambda i: (0, i, 0)),
            pl.BlockSpec((_BR, D_HID), lambda i: (i, 0)),
            pl.BlockSpec((D_HID, D_OUT), lambda i: (0, 0)),
            pl.BlockSpec((D_HID, D_OUT), lambda i: (0, 0)),
            pl.BlockSpec((1, D_OUT), lambda i: (0, 0)),
        ],
        out_specs=pl.BlockSpec((_BR, D_OUT), lambda i: (i, 0)),
        out_shape=jax.ShapeDtypeStruct((N_NODES, D_OUT), jnp.float32),
    )(parts2, cntp, h, w2lT, w2rT, b2)


def kernel(x, edge_index, W1l, b1, W1r, W2l, b2, W2r):
    src = jnp.pad(edge_index[0].astype(jnp.int32), (0, E_PAD - N_EDGES),
                  constant_values=ZROW).reshape(NW, NCHUNK, CH)
    dst = jnp.pad(edge_index[1].astype(jnp.int32), (0, E_PAD - N_EDGES),
                  constant_values=TRASH).reshape(NW, NCHUNK, CH)

    z32 = jnp.zeros((NP_, 32), jnp.float32)
    z16 = jnp.zeros((NP_, 16), jnp.float32)
    ones = jnp.ones((CH, 16), jnp.float32)

    w1catT = jnp.concatenate([W1l, W1r], axis=0).T
    y = _tc1(x, w1catT)
    table1 = jnp.pad(y[:, :D_HID], ((0, NP_ - N_NODES), (0, 0)))
    r1 = y[:, D_HID:]

    parts1, cntp = _sc_pass1(table1, src, dst, z32, z16, ones)

    h = _tc2(parts1, cntp, r1, b1.reshape(1, D_HID))
    table2 = jnp.pad(h, ((0, NP_ - N_NODES), (0, 0)))

    (parts2,) = _sc_pass2(table2, src, dst, z32)

    return _tc3(parts2, cntp, h, W2l.T, W2r.T, b2.reshape(1, D_OUT))


# speedup vs baseline: 20.5655x; 2.1756x over previous
import jax
import jax.numpy as jnp
from jax import lax
from jax.experimental import pallas as pl
from jax.experimental.pallas import tpu as pltpu
from jax.experimental.pallas import tpu_sc as plsc

N_NODES = 10000
N_EDGES = 320000
D_IN = 128
D_HID = 32
D_OUT = 128

NP_ = 10112

NC, NS = 2, 16
CH = 128
NCH = N_EDGES // CH
KB = 78
KMAX = KB + 1
NPAIR = KB // 2

R = NP_ // NS

_mesh = plsc.VectorSubcoreMesh(
    core_axis_name="c", subcore_axis_name="s", num_cores=NC, num_subcores=NS)


def _make_sc_pass(with_counts: bool):
    out_type = [jax.ShapeDtypeStruct((NC, NP_, 32), jnp.float32)]
    if with_counts:
        out_type.append(jax.ShapeDtypeStruct((NC, NP_, 16), jnp.float32))

    scratch = [
        pltpu.VMEM_SHARED((NP_, 32), jnp.float32),
        pltpu.VMEM_SHARED((NP_, 32), jnp.float32),
        pltpu.VMEM((KMAX, CH), jnp.int32),
        pltpu.VMEM((KMAX, CH), jnp.int32),
        pltpu.VMEM((CH, 32), jnp.float32),
        pltpu.VMEM((CH, 32), jnp.float32),
        pltpu.SemaphoreType.DMA,
        pltpu.SemaphoreType.DMA,
        pltpu.SemaphoreType.DMA,
        pltpu.SemaphoreType.DMA,
    ]
    if with_counts:
        scratch += [
            pltpu.VMEM_SHARED((NP_, 16), jnp.float32),
            pltpu.VMEM((CH, 16), jnp.float32),
            pltpu.SemaphoreType.DMA,
            pltpu.SemaphoreType.DMA,
        ]

    def body(*refs):
        if with_counts:
            (table_h, e_h, z32_h, z16_h, ones_h,
             parts_h, cntp_h,
             table_s, acc_s, src_v, dst_v, rows0, rows1, gs0, gs1, ss0, ss1,
             cnt_s, ones_v, cs0, cs1) = refs
        else:
            (table_h, e_h, z32_h,
             parts_h,
             table_s, acc_s, src_v, dst_v, rows0, rows1, gs0, gs1, ss0, ss1) = refs
        c = lax.axis_index("c")
        s = lax.axis_index("s")
        base = jnp.where(c == 0, s * KB,
                         NS * KB + s * KB + jnp.maximum(s - 12, 0))
        extra = jnp.logical_and(c == 1, s >= 12)

        pltpu.sync_copy(table_h.at[pl.ds(s * R, R)], table_s.at[pl.ds(s * R, R)])
        pltpu.sync_copy(z32_h.at[pl.ds(s * R, R)], acc_s.at[pl.ds(s * R, R)])
        if with_counts:
            pltpu.sync_copy(z16_h.at[pl.ds(s * R, R)], cnt_s.at[pl.ds(s * R, R)])
            pltpu.sync_copy(ones_h, ones_v)
        pltpu.sync_copy(e_h.at[0, pl.ds(base, KMAX)], src_v)
        pltpu.sync_copy(e_h.at[1, pl.ds(base, KMAX)], dst_v)
        plsc.subcore_barrier()

        def gather(j, buf, sem):
            return pltpu.async_copy(table_s.at[src_v.at[j]], buf, sem)

        def scat(j, buf, sem):
            return pltpu.async_copy(buf, acc_s.at[dst_v.at[j]], sem, add=True)

        def cnts(j, sem):
            return pltpu.async_copy(ones_v, cnt_s.at[dst_v.at[j]], sem, add=True)

        gather(0, rows0, gs0)

        def pair(i, carry):
            j0 = 2 * i
            j1 = j0 + 1
            pltpu.make_async_copy(table_s.at[src_v.at[j0]], rows0, gs0).wait()

            @pl.when(i > 0)
            def _():
                pltpu.make_async_copy(
                    rows1, acc_s.at[dst_v.at[j0]], ss1).wait()
                if with_counts:
                    pltpu.make_async_copy(
                        ones_v, cnt_s.at[dst_v.at[j0]], cs1).wait()

            g1 = gather(j1, rows1, gs1)
            s0 = scat(j0, rows0, ss0)
            c0 = cnts(j0, cs0) if with_counts else None
            g1.wait()
            s0.wait()
            if with_counts:
                c0.wait()

            @pl.when(i < NPAIR - 1)
            def _():
                gather(j0 + 2, rows0, gs0)

            scat(j1, rows1, ss1)
            if with_counts:
                cnts(j1, cs1)
            return carry

        lax.fori_loop(0, NPAIR, pair, 0)

        @pl.when(extra)
        def _():
            gather(KB, rows0, gs0).wait()
            st = scat(KB, rows0, ss0)
            if with_counts:
                cnts(KB, cs0).wait()
            st.wait()

        pltpu.make_async_copy(rows1, acc_s.at[dst_v.at[KB - 1]], ss1).wait()
        if with_counts:
            pltpu.make_async_copy(
                ones_v, cnt_s.at[dst_v.at[KB - 1]], cs1).wait()
        plsc.subcore_barrier()

        pltpu.sync_copy(acc_s.at[pl.ds(s * R, R)], parts_h.at[c, pl.ds(s * R, R)])
        if with_counts:
            pltpu.sync_copy(cnt_s.at[pl.ds(s * R, R)], cntp_h.at[c, pl.ds(s * R, R)])

    return pl.kernel(body, out_type=out_type, mesh=_mesh, scratch_types=scratch,
                     compiler_params=pltpu.CompilerParams(use_tc_tiling_on_sc=False))


_sc_pass1 = _make_sc_pass(with_counts=True)
_sc_pass2 = _make_sc_pass(with_counts=False)


_BR = NP_ // 4
_BR3 = _BR


def _rowmask(blk):
    rows = (pl.program_id(0) * blk
            + lax.broadcasted_iota(jnp.int32, (blk, 1), 0))
    return rows < N_NODES


def _tc1_body(x_ref, wl_ref, wr_ref, t_ref, r_ref):
    m = _rowmask(_BR)
    x = x_ref[...]
    t_ref[...] = jnp.where(
        m, jnp.dot(x, wl_ref[...], preferred_element_type=jnp.float32), 0.0)
    r_ref[...] = jnp.where(
        m, jnp.dot(x, wr_ref[...], preferred_element_type=jnp.float32), 0.0)


def _tc1(x, w1lT, w1rT):
    return pl.pallas_call(
        _tc1_body,
        grid=(4,),
        in_specs=[
            pl.BlockSpec((_BR, D_IN), lambda i: (i, 0)),
            pl.BlockSpec((D_IN, D_HID), lambda i: (0, 0)),
            pl.BlockSpec((D_IN, D_HID), lambda i: (0, 0)),
        ],
        out_specs=[
            pl.BlockSpec((_BR, D_HID), lambda i: (i, 0)),
            pl.BlockSpec((_BR, D_HID), lambda i: (i, 0)),
        ],
        out_shape=[
            jax.ShapeDtypeStruct((NP_, D_HID), jnp.float32),
            jax.ShapeDtypeStruct((NP_, D_HID), jnp.float32),
        ],
    )(x, w1lT, w1rT)


def _tc2_body(p_ref, cp_ref, r_ref, b_ref, h_ref):
    psum = p_ref[0] + p_ref[1]
    cnt = cp_ref[0, :, 0:1] + cp_ref[1, :, 0:1]
    inv = 1.0 / jnp.maximum(cnt, 1.0)
    h = jnp.maximum(psum * inv + b_ref[...] + r_ref[...], 0.0)
    h_ref[...] = jnp.where(_rowmask(_BR), h, 0.0)


def _tc2(parts1, cntp, r1, b1):
    return pl.pallas_call(
        _tc2_body,
        grid=(4,),
        in_specs=[
            pl.BlockSpec((NC, _BR, D_HID), lambda i: (0, i, 0)),
            pl.BlockSpec((NC, _BR, 16), lambda i: (0, i, 0)),
            pl.BlockSpec((_BR, D_HID), lambda i: (i, 0)),
            pl.BlockSpec((1, D_HID), lambda i: (0, 0)),
        ],
        out_specs=pl.BlockSpec((_BR, D_HID), lambda i: (i, 0)),
        out_shape=jax.ShapeDtypeStruct((NP_, D_HID), jnp.float32),
    )(parts1, cntp, r1, b1)


def _tc3_body(p_ref, cp_ref, h_ref, wl_ref, wr_ref, b_ref, o_ref):
    cnt = cp_ref[0, :, 0:1] + cp_ref[1, :, 0:1]
    inv = 1.0 / jnp.maximum(cnt, 1.0)
    agg = (p_ref[0] + p_ref[1]) * inv
    o_ref[...] = (
        jnp.dot(agg, wl_ref[...], preferred_element_type=jnp.float32)
        + jnp.dot(h_ref[...], wr_ref[...], preferred_element_type=jnp.float32)
        + b_ref[...])


def _tc3(parts2, cntp, h, w2lT, w2rT, b2):
    return pl.pallas_call(
        _tc3_body,
        grid=(4,),
        in_specs=[
            pl.BlockSpec((NC, _BR3, D_HID), lambda i: (0, i, 0)),
            pl.BlockSpec((NC, _BR3, 16), lambda i: (0, i, 0)),
            pl.BlockSpec((_BR3, D_HID), lambda i: (i, 0)),
            pl.BlockSpec((D_HID, D_OUT), lambda i: (0, 0)),
            pl.BlockSpec((D_HID, D_OUT), lambda i: (0, 0)),
            pl.BlockSpec((1, D_OUT), lambda i: (0, 0)),
        ],
        out_specs=pl.BlockSpec((_BR3, D_OUT), lambda i: (i, 0)),
        out_shape=jax.ShapeDtypeStruct((N_NODES, D_OUT), jnp.float32),
    )(parts2, cntp, h, w2lT, w2rT, b2)


def kernel(x, edge_index, W1l, b1, W1r, W2l, b2, W2r):
    e3 = edge_index.astype(jnp.int32).reshape(2, NCH, CH)

    z32 = jnp.zeros((NP_, 32), jnp.float32)
    z16 = jnp.zeros((NP_, 16), jnp.float32)
    ones = jnp.ones((CH, 16), jnp.float32)

    table1, r1 = _tc1(x, W1l.T, W1r.T)

    parts1, cntp = _sc_pass1(table1, e3, z32, z16, ones)

    h = _tc2(parts1, cntp, r1, b1.reshape(1, D_HID))

    (parts2,) = _sc_pass2(h, e3, z32)

    return _tc3(parts2, cntp, h, W2l.T, W2r.T, b2.reshape(1, D_OUT))
